# Initial kernel scaffold; baseline (speedup 1.0000x reference)
import jax
import jax.numpy as jnp
from jax.experimental import pallas as pl


def kernel(x, w1, g1, b1, w2):
    raise NotImplementedError("write your pallas kernel here")


# speedup vs baseline: 1.4698x; 1.4698x over previous
import functools

import jax
import jax.numpy as jnp
from jax import lax
from jax.experimental import pallas as pl
from jax.experimental.pallas import tpu as pltpu

_EPS = 1e-5
_NCORES = 2


def _h_stats_kernel(x_ref, w1_ref, h_ref, s1_ref):
    b = pl.program_id(1)

    @pl.when(b == 0)
    def _init():
        s1_ref[...] = jnp.zeros_like(s1_ref)

    xb = x_ref[...].astype(jnp.bfloat16)
    w1b = w1_ref[...].astype(jnp.bfloat16)
    h = jnp.dot(xb, w1b, preferred_element_type=jnp.float32)
    h_ref[...] = h.astype(jnp.bfloat16)
    s1_ref[0:1, :] += jnp.sum(h, axis=0, keepdims=True)
    s1_ref[1:2, :] += jnp.sum(h * h, axis=0, keepdims=True)


def _gram_kernel(h_ref, g1_ref, b1_ref, s1p_ref, gr_ref, sr_ref, *, batch):
    b = pl.program_id(1)

    @pl.when(b == 0)
    def _init():
        gr_ref[...] = jnp.zeros_like(gr_ref)
        sr_ref[...] = jnp.zeros_like(sr_ref)

    inv_b = 1.0 / batch
    ssum = s1p_ref[0:1, :] + s1p_ref[8:9, :]
    ssq = s1p_ref[1:2, :] + s1p_ref[9:10, :]
    m1 = ssum * inv_b
    var1 = ssq * inv_b - m1 * m1
    scale = lax.rsqrt(var1 + _EPS) * g1_ref[...]
    shift = b1_ref[...] - m1 * scale

    h = h_ref[...].astype(jnp.float32)
    r = jnp.maximum(h * scale + shift, 0.0)
    rb = r.astype(jnp.bfloat16)
    gr_ref[...] += lax.dot_general(
        rb, rb, (((0,), (0,)), ((), ())), preferred_element_type=jnp.float32)
    sr_ref[0:1, :] += jnp.sum(r, axis=0, keepdims=True)


def _finalize_kernel(s1p_ref, grp_ref, srp_ref, w2_ref, s1_ref, s2_ref,
                     *, batch, feat):
    inv_b = 1.0 / batch
    ssum = s1p_ref[0:1, :] + s1p_ref[8:9, :]
    ssq = s1p_ref[1:2, :] + s1p_ref[9:10, :]
    m1 = ssum * inv_b
    var1 = ssq * inv_b - m1 * m1
    s1_ref[0:1, :] = m1
    s1_ref[1:2, :] = lax.rsqrt(var1 + _EPS)

    w2 = w2_ref[...]
    g = grp_ref[0:feat, :] + grp_ref[feat:2 * feat, :]
    sr = srp_ref[0:1, :] + srp_ref[8:9, :]
    m2 = jnp.dot(sr, w2, preferred_element_type=jnp.float32,
                 precision=lax.Precision.HIGHEST) * inv_b
    t = jnp.dot(g, w2, preferred_element_type=jnp.float32,
                precision=lax.Precision.HIGHEST)
    e2 = jnp.sum(w2 * t, axis=0, keepdims=True) * inv_b
    var2 = e2 - m2 * m2
    s2_ref[0:1, :] = m2
    s2_ref[1:2, :] = lax.rsqrt(var2 + _EPS)


def _out_kernel(h_ref, g1_ref, b1_ref, w2_ref, s1_ref, s2_ref, o_ref):
    scale = s1_ref[1:2, :] * g1_ref[...]
    shift = b1_ref[...] - s1_ref[0:1, :] * scale
    h = h_ref[...].astype(jnp.float32)
    r = jnp.maximum(h * scale + shift, 0.0)
    w2b = w2_ref[...].astype(jnp.bfloat16)
    y = jnp.dot(r.astype(jnp.bfloat16), w2b,
                preferred_element_type=jnp.float32)
    o_ref[...] = ((y - s2_ref[0:1, :]) * s2_ref[1:2, :]).astype(o_ref.dtype)


def kernel(x, w1, g1, b1, w2):
    B, D = x.shape
    P = w2.shape[1]

    tb = None
    for cand in (8192, 4096, 2048, 1024, 512, 256, 128, 8):
        if B % (_NCORES * cand) == 0:
            tb = cand
            break
    nbc = B // (_NCORES * tb)
    nb = _NCORES * nbc

    h, s1p = pl.pallas_call(
        _h_stats_kernel,
        out_shape=(jax.ShapeDtypeStruct((B, D), jnp.bfloat16),
                   jax.ShapeDtypeStruct((_NCORES * 8, D), jnp.float32)),
        grid=(_NCORES, nbc),
        in_specs=[
            pl.BlockSpec((tb, D), lambda c, b: (c * nbc + b, 0)),
            pl.BlockSpec((D, D), lambda c, b: (0, 0)),
        ],
        out_specs=(
            pl.BlockSpec((tb, D), lambda c, b: (c * nbc + b, 0)),
            pl.BlockSpec((8, D), lambda c, b: (c, 0)),
        ),
        compiler_params=pltpu.CompilerParams(
            dimension_semantics=("parallel", "arbitrary")),
    )(x, w1)

    gram_kernel = functools.partial(_gram_kernel, batch=B)
    grp, srp = pl.pallas_call(
        gram_kernel,
        out_shape=(jax.ShapeDtypeStruct((_NCORES * D, D), jnp.float32),
                   jax.ShapeDtypeStruct((_NCORES * 8, D), jnp.float32)),
        grid=(_NCORES, nbc),
        in_specs=[
            pl.BlockSpec((tb, D), lambda c, b: (c * nbc + b, 0)),
            pl.BlockSpec((1, D), lambda c, b: (0, 0)),
            pl.BlockSpec((1, D), lambda c, b: (0, 0)),
            pl.BlockSpec((_NCORES * 8, D), lambda c, b: (0, 0)),
        ],
        out_specs=(
            pl.BlockSpec((D, D), lambda c, b: (c, 0)),
            pl.BlockSpec((8, D), lambda c, b: (c, 0)),
        ),
        compiler_params=pltpu.CompilerParams(
            dimension_semantics=("parallel", "arbitrary")),
    )(h, g1, b1, s1p)

    finalize_kernel = functools.partial(_finalize_kernel, batch=B, feat=D)
    s1, s2 = pl.pallas_call(
        finalize_kernel,
        out_shape=(jax.ShapeDtypeStruct((8, D), jnp.float32),
                   jax.ShapeDtypeStruct((8, P), jnp.float32)),
    )(s1p, grp, srp, w2)

    out = pl.pallas_call(
        _out_kernel,
        out_shape=jax.ShapeDtypeStruct((B, P), x.dtype),
        grid=(nb,),
        in_specs=[
            pl.BlockSpec((tb, D), lambda b: (b, 0)),
            pl.BlockSpec((1, D), lambda b: (0, 0)),
            pl.BlockSpec((1, D), lambda b: (0, 0)),
            pl.BlockSpec((D, P), lambda b: (0, 0)),
            pl.BlockSpec((8, D), lambda b: (0, 0)),
            pl.BlockSpec((8, P), lambda b: (0, 0)),
        ],
        out_specs=pl.BlockSpec((tb, P), lambda b: (b, 0)),
        compiler_params=pltpu.CompilerParams(
            dimension_semantics=("parallel",)),
    )(h, g1, b1, w2, s1, s2)
    return out
